# baseline (speedup 1.0000x reference)
import functools

import jax
import jax.numpy as jnp
from jax import lax
from jax.experimental import pallas as pl
from jax.experimental.pallas import tpu as pltpu
from jax.experimental.pallas import tpu_sc as plsc

NUM_CORES = 2
NUM_SUBCORES = 16
NUM_WORKERS = NUM_CORES * NUM_SUBCORES
NUM_CHUNKS = 8
NBUF = 4
LANES = 16


def _make_gather(vocab2, dim, batch):
    assert batch % (NUM_WORKERS * NUM_CHUNKS) == 0
    b_per_w = batch // NUM_WORKERS
    chunk = b_per_w // NUM_CHUNKS

    mesh = plsc.VectorSubcoreMesh(core_axis_name="c", subcore_axis_name="s")

    @functools.partial(
        pl.kernel,
        mesh=mesh,
        out_type=jax.ShapeDtypeStruct((batch, dim), jnp.float32),
        scratch_types=[
            pltpu.VMEM((b_per_w,), jnp.int32),
            pltpu.VMEM((b_per_w,), jnp.int32),
            pltpu.VMEM((NBUF * (b_per_w // NUM_CHUNKS), 2 * dim),
                       jnp.float32),
            pltpu.VMEM((b_per_w, dim), jnp.float32),
            [pltpu.SemaphoreType.DMA] * NBUF,
            [pltpu.SemaphoreType.DMA] * NUM_CHUNKS,
        ],
        compiler_params=pltpu.CompilerParams(),
    )
    def gather_kernel(table_hbm, idx_hbm, out_hbm, idsv, qv, buf, outb,
                      gsems, osems):
        wid = lax.axis_index("s") * NUM_CORES + lax.axis_index("c")
        base = wid * b_per_w
        pltpu.sync_copy(idx_hbm.at[pl.ds(base, b_per_w)], idsv)

        for i in range(b_per_w // LANES):
            sl = pl.ds(i * LANES, LANES)
            qv[sl] = lax.shift_right_logical(idsv[sl], 1)

        def fire_gather(c):
            s = c % NBUF
            return pltpu.async_copy(
                table_hbm.at[qv.at[pl.ds(c * chunk, chunk)]],
                buf.at[pl.ds(s * chunk, chunk)],
                gsems[s],
            )

        gathers = [fire_gather(c) for c in range(NBUF)]

        writes = []
        for c in range(NUM_CHUNKS):
            s = c % NBUF
            gathers[s].wait()

            def body(g, _):
                r0 = g * LANES
                offs = lax.shift_left(
                    idsv[pl.ds(c * chunk + r0, LANES)] & 1, 6)
                for j in range(LANES):
                    src_r = s * chunk + r0 + j
                    dst_r = c * chunk + r0 + j
                    for k in range(dim // LANES):
                        outb[dst_r, pl.ds(k * LANES, LANES)] = buf[
                            src_r, pl.ds(offs[j] + k * LANES, LANES)
                        ]
                return _

            lax.fori_loop(0, chunk // LANES, body, None)
            if c + NBUF < NUM_CHUNKS:
                gathers[s] = fire_gather(c + NBUF)
            writes.append(
                pltpu.async_copy(
                    outb.at[pl.ds(c * chunk, chunk)],
                    out_hbm.at[pl.ds(base + c * chunk, chunk)],
                    osems[c],
                )
            )
        for w in writes:
            w.wait()

    return gather_kernel


def kernel(topology_ids, embedding_table):
    vocab, dim = embedding_table.shape
    (batch,) = topology_ids.shape
    table2 = embedding_table.reshape(vocab // 2, 2 * dim)
    gather = _make_gather(vocab // 2, dim, batch)
    return gather(table2, topology_ids.astype(jnp.int32))


# speedup vs baseline: 1.0020x; 1.0020x over previous
import functools

import jax
import jax.numpy as jnp
from jax import lax
from jax.experimental import pallas as pl
from jax.experimental.pallas import tpu as pltpu
from jax.experimental.pallas import tpu_sc as plsc

NUM_CORES = 2
NUM_SUBCORES = 16
NUM_WORKERS = NUM_CORES * NUM_SUBCORES
NUM_CHUNKS = 8
LANES = 16


def _make_gather(vocab, dim, batch):
    assert batch % (NUM_WORKERS * NUM_CHUNKS) == 0
    b_per_w = batch // NUM_WORKERS
    chunk = b_per_w // NUM_CHUNKS

    mesh = plsc.VectorSubcoreMesh(core_axis_name="c", subcore_axis_name="s")

    @functools.partial(
        pl.kernel,
        mesh=mesh,
        out_type=jax.ShapeDtypeStruct((batch * dim // 128, 128),
                                      jnp.float32),
        scratch_types=[
            pltpu.VMEM((b_per_w,), jnp.int32),
            pltpu.VMEM((b_per_w, dim), jnp.float32),
            pltpu.VMEM((b_per_w * dim // 128, 128), jnp.float32),
            [pltpu.SemaphoreType.DMA] * NUM_CHUNKS,
            [pltpu.SemaphoreType.DMA] * NUM_CHUNKS,
        ],
        compiler_params=pltpu.CompilerParams(use_tc_tiling_on_sc=False),
    )
    def gather_kernel(table_hbm, idx_hbm, out_hbm, idsv, rows_v, packed,
                      gsems, osems):
        wid = lax.axis_index("s") * NUM_CORES + lax.axis_index("c")
        base = wid * b_per_w
        pltpu.sync_copy(idx_hbm.at[pl.ds(base, b_per_w)], idsv)

        gathers = [
            pltpu.async_copy(
                table_hbm.at[idsv.at[pl.ds(c * chunk, chunk)]],
                rows_v.at[pl.ds(c * chunk, chunk)],
                gsems[c],
            )
            for c in range(NUM_CHUNKS)
        ]

        rows_per_out = 128 // dim
        writes = []
        for c in range(NUM_CHUNKS):
            gathers[c].wait()

            def body(i, _):
                src0 = c * chunk + i * rows_per_out
                dst = c * chunk // rows_per_out + i
                for h in range(rows_per_out):
                    for k in range(dim // LANES):
                        packed[dst, pl.ds(h * dim + k * LANES, LANES)] = (
                            rows_v[src0 + h, pl.ds(k * LANES, LANES)]
                        )
                return _

            lax.fori_loop(0, chunk // rows_per_out, body, None)
            writes.append(
                pltpu.async_copy(
                    packed.at[pl.ds(c * chunk // rows_per_out,
                                    chunk // rows_per_out)],
                    out_hbm.at[pl.ds((base + c * chunk) // rows_per_out,
                                     chunk // rows_per_out)],
                    osems[c],
                )
            )
        for w in writes:
            w.wait()

    return gather_kernel


def kernel(topology_ids, embedding_table):
    vocab, dim = embedding_table.shape
    (batch,) = topology_ids.shape
    gather = _make_gather(vocab, dim, batch)
    wide = gather(embedding_table, topology_ids.astype(jnp.int32))
    return wide.reshape(batch, dim)
